# baseline (speedup 1.0000x reference)
import jax
import jax.numpy as jnp
from jax import lax
from jax.experimental import pallas as pl
from jax.experimental.pallas import tpu as pltpu
from jax.experimental.pallas import tpu_sc as plsc

_B, _N, _K = 64, 32768, 2048
_NC, _NS = 2, 16
_NW = _NC * _NS
_RW = _B // _NW
_L = 16
_STEPS = _K // _L


def _gather_body(x_hbm, idx_hbm, out_hbm, x_v, i_v, o_v, sem_a, sem_b):
    wid = lax.axis_index("s") * _NC + lax.axis_index("c")
    row0 = wid * _RW
    row1 = row0 + 1

    di0 = pltpu.async_copy(idx_hbm.at[row0], i_v.at[pl.ds(0, _K)], sem_a)
    di1 = pltpu.async_copy(idx_hbm.at[row1], i_v.at[pl.ds(_K, _K)], sem_a)
    dx0 = pltpu.async_copy(x_hbm.at[row0], x_v.at[pl.ds(_N, _N)], sem_a)
    dx1 = pltpu.async_copy(x_hbm.at[row1], x_v.at[pl.ds(0, _N)], sem_b)

    def gather_row(base, idx_bias):
        def step(i, carry):
            iv = i_v[pl.ds(base + i * _L, _L)]
            out_v = plsc.load_gather(x_v, [iv + idx_bias] if idx_bias else [iv])
            o_v[pl.ds(base + i * _L, _L)] = out_v
            return carry

        lax.fori_loop(0, _STEPS, step, 0, unroll=16)

    di0.wait()
    di1.wait()
    dx0.wait()
    gather_row(0, _N)
    do0 = pltpu.async_copy(o_v.at[pl.ds(0, _K)], out_hbm.at[row0], sem_a)
    dx1.wait()
    gather_row(_K, 0)
    do1 = pltpu.async_copy(o_v.at[pl.ds(_K, _K)], out_hbm.at[row1], sem_b)
    do0.wait()
    do1.wait()


@jax.jit
def _run(x, idx):
    mesh = plsc.VectorSubcoreMesh(core_axis_name="c", subcore_axis_name="s")
    f = pl.kernel(
        _gather_body,
        out_type=jax.ShapeDtypeStruct((_B, _K), jnp.float32),
        mesh=mesh,
        scratch_types=[
            pltpu.VMEM((_RW * _N,), jnp.float32),
            pltpu.VMEM((_RW * _K,), jnp.int32),
            pltpu.VMEM((_RW * _K,), jnp.float32),
            pltpu.SemaphoreType.DMA,
            pltpu.SemaphoreType.DMA,
        ],
        compiler_params=pltpu.CompilerParams(needs_layout_passes=False),
    )
    return f(x, idx)


def kernel(x, assessment, index):
    del assessment
    return _run(x, index.astype(jnp.int32))


# speedup vs baseline: 1.0126x; 1.0126x over previous
import jax
import jax.numpy as jnp
from jax import lax
from jax.experimental import pallas as pl
from jax.experimental.pallas import tpu as pltpu
from jax.experimental.pallas import tpu_sc as plsc

_B, _N, _K = 64, 32768, 2048
_NC, _NS = 2, 16
_NW = _NC * _NS
_RW = _B // _NW
_L = 16
_STEPS = _K // _L


def _gather_body(x_hbm, idx_hbm, out_hbm,
                 x0_v, x1_v, i0_v, i1_v, o0_v, o1_v,
                 sem_a, sem_b, sem_o):
    wid = lax.axis_index("s") * _NC + lax.axis_index("c")
    row0 = wid * _RW
    row1 = row0 + 1

    di0 = pltpu.async_copy(idx_hbm.at[row0], i0_v, sem_a)
    di1 = pltpu.async_copy(idx_hbm.at[row1], i1_v, sem_b)
    dx0 = pltpu.async_copy(x_hbm.at[row0], x0_v, sem_a)
    dx1 = pltpu.async_copy(x_hbm.at[row1], x1_v, sem_b)

    def gather_row(x_v, idx_v, out_v):
        def step(i, carry):
            iv = idx_v[pl.ds(i * _L, _L)]
            out_v[pl.ds(i * _L, _L)] = plsc.load_gather(x_v, [iv])
            return carry

        lax.fori_loop(0, _STEPS, step, 0, unroll=16)

    di0.wait()
    dx0.wait()
    gather_row(x0_v, i0_v, o0_v)
    do0 = pltpu.async_copy(o0_v, out_hbm.at[row0], sem_o)
    di1.wait()
    dx1.wait()
    gather_row(x1_v, i1_v, o1_v)
    do1 = pltpu.async_copy(o1_v, out_hbm.at[row1], sem_o)
    do0.wait()
    do1.wait()


@jax.jit
def _run(x, idx):
    mesh = plsc.VectorSubcoreMesh(core_axis_name="c", subcore_axis_name="s")
    f = pl.kernel(
        _gather_body,
        out_type=jax.ShapeDtypeStruct((_B, _K), jnp.float32),
        mesh=mesh,
        scratch_types=[
            pltpu.VMEM((_N,), jnp.float32),
            pltpu.VMEM((_N,), jnp.float32),
            pltpu.VMEM((_K,), jnp.int32),
            pltpu.VMEM((_K,), jnp.int32),
            pltpu.VMEM((_K,), jnp.float32),
            pltpu.VMEM((_K,), jnp.float32),
            pltpu.SemaphoreType.DMA,
            pltpu.SemaphoreType.DMA,
            pltpu.SemaphoreType.DMA,
        ],
        compiler_params=pltpu.CompilerParams(needs_layout_passes=False),
    )
    return f(x, idx)


def kernel(x, assessment, index):
    del assessment
    return _run(x, index.astype(jnp.int32))


# speedup vs baseline: 1.0628x; 1.0496x over previous
import jax
import jax.numpy as jnp
from jax import lax
from jax.experimental import pallas as pl
from jax.experimental.pallas import tpu as pltpu
from jax.experimental.pallas import tpu_sc as plsc

_B, _N, _K = 64, 32768, 2048
_NC, _NS = 2, 16
_NW = _NC * _NS
_RW = _B // _NW
_L = 16
_STEPS = _K // _L


def _gather_body(x_hbm, idx_hbm, out_hbm,
                 x0_v, x1_v, i0_v, i1_v, o0_v, o1_v,
                 sem_a, sem_b, sem_o):
    wid = lax.axis_index("s") * _NC + lax.axis_index("c")
    row0 = wid * _RW
    row1 = row0 + 1

    di0 = pltpu.async_copy(idx_hbm.at[row0], i0_v, sem_a)
    di1 = pltpu.async_copy(idx_hbm.at[row1], i1_v, sem_b)
    dx0 = pltpu.async_copy(x_hbm.at[row0], x0_v, sem_a)
    dx1 = pltpu.async_copy(x_hbm.at[row1], x1_v, sem_b)

    def gather_row(x_v, idx_v, out_v):
        @plsc.parallel_loop(0, _K, step=_L, unroll=16)
        def step(i):
            iv = idx_v[pl.ds(i, _L)]
            out_v[pl.ds(i, _L)] = plsc.load_gather(x_v, [iv])

    di0.wait()
    dx0.wait()
    gather_row(x0_v, i0_v, o0_v)
    do0 = pltpu.async_copy(o0_v, out_hbm.at[row0], sem_o)
    di1.wait()
    dx1.wait()
    gather_row(x1_v, i1_v, o1_v)
    do1 = pltpu.async_copy(o1_v, out_hbm.at[row1], sem_o)
    do0.wait()
    do1.wait()


@jax.jit
def _run(x, idx):
    mesh = plsc.VectorSubcoreMesh(core_axis_name="c", subcore_axis_name="s")
    f = pl.kernel(
        _gather_body,
        out_type=jax.ShapeDtypeStruct((_B, _K), jnp.float32),
        mesh=mesh,
        scratch_types=[
            pltpu.VMEM((_N,), jnp.float32),
            pltpu.VMEM((_N,), jnp.float32),
            pltpu.VMEM((_K,), jnp.int32),
            pltpu.VMEM((_K,), jnp.int32),
            pltpu.VMEM((_K,), jnp.float32),
            pltpu.VMEM((_K,), jnp.float32),
            pltpu.SemaphoreType.DMA,
            pltpu.SemaphoreType.DMA,
            pltpu.SemaphoreType.DMA,
        ],
        compiler_params=pltpu.CompilerParams(needs_layout_passes=False),
    )
    return f(x, idx)


def kernel(x, assessment, index):
    del assessment
    return _run(x, index.astype(jnp.int32))


# speedup vs baseline: 1.0772x; 1.0136x over previous
import jax
import jax.numpy as jnp
from jax import lax
from jax.experimental import pallas as pl
from jax.experimental.pallas import tpu as pltpu
from jax.experimental.pallas import tpu_sc as plsc

_B, _N, _K = 64, 32768, 2048
_NC, _NS = 2, 16
_NW = _NC * _NS
_RW = _B // _NW
_L = 16
_STEPS = _K // _L


def _gather_body(x_hbm, idx_hbm, out_hbm,
                 x0_v, x1_v, i0_v, i1_v, o0_v, o1_v,
                 sem_a, sem_b, sem_o):
    wid = lax.axis_index("s") * _NC + lax.axis_index("c")
    row0 = wid * _RW
    row1 = row0 + 1

    di0 = pltpu.async_copy(idx_hbm.at[row0], i0_v, sem_a)
    di1 = pltpu.async_copy(idx_hbm.at[row1], i1_v, sem_b)
    dx0 = pltpu.async_copy(x_hbm.at[row0], x0_v, sem_a)
    dx1 = pltpu.async_copy(x_hbm.at[row1], x1_v, sem_b)

    def gather_row(x_v, idx_v, out_v):
        @plsc.parallel_loop(0, _K, step=_L, unroll=32)
        def step(i):
            iv = idx_v[pl.ds(i, _L)]
            out_v[pl.ds(i, _L)] = plsc.load_gather(x_v, [iv])

    di0.wait()
    dx0.wait()
    gather_row(x0_v, i0_v, o0_v)
    do0 = pltpu.async_copy(o0_v, out_hbm.at[row0], sem_o)
    di1.wait()
    dx1.wait()
    gather_row(x1_v, i1_v, o1_v)
    do1 = pltpu.async_copy(o1_v, out_hbm.at[row1], sem_o)
    do0.wait()
    do1.wait()


@jax.jit
def _run(x, idx):
    mesh = plsc.VectorSubcoreMesh(core_axis_name="c", subcore_axis_name="s")
    f = pl.kernel(
        _gather_body,
        out_type=jax.ShapeDtypeStruct((_B, _K), jnp.float32),
        mesh=mesh,
        scratch_types=[
            pltpu.VMEM((_N,), jnp.float32),
            pltpu.VMEM((_N,), jnp.float32),
            pltpu.VMEM((_K,), jnp.int32),
            pltpu.VMEM((_K,), jnp.int32),
            pltpu.VMEM((_K,), jnp.float32),
            pltpu.VMEM((_K,), jnp.float32),
            pltpu.SemaphoreType.DMA,
            pltpu.SemaphoreType.DMA,
            pltpu.SemaphoreType.DMA,
        ],
        compiler_params=pltpu.CompilerParams(needs_layout_passes=False),
    )
    return f(x, idx)


def kernel(x, assessment, index):
    del assessment
    return _run(x, index.astype(jnp.int32))


# speedup vs baseline: 1.0860x; 1.0081x over previous
import jax
import jax.numpy as jnp
from jax import lax
from jax.experimental import pallas as pl
from jax.experimental.pallas import tpu as pltpu
from jax.experimental.pallas import tpu_sc as plsc

_B, _N, _K = 64, 32768, 2048
_NC, _NS = 2, 16
_NW = _NC * _NS
_RW = _B // _NW
_L = 16
_STEPS = _K // _L


def _gather_body(x_hbm, idx_hbm, out_hbm,
                 x0_v, x1_v, i0_v, i1_v, o0_v, o1_v,
                 sem_a, sem_b, sem_o):
    wid = lax.axis_index("s") * _NC + lax.axis_index("c")
    row0 = wid * _RW
    row1 = row0 + 1

    di0 = pltpu.async_copy(idx_hbm.at[row0], i0_v, sem_a)
    dx0 = pltpu.async_copy(x_hbm.at[row0], x0_v, sem_a)
    di1 = pltpu.async_copy(idx_hbm.at[row1], i1_v, sem_b)
    dx1 = pltpu.async_copy(x_hbm.at[row1], x1_v, sem_b)

    def gather_row(x_v, idx_v, out_v):
        @plsc.parallel_loop(0, _K, step=_L, unroll=16)
        def step(i):
            iv = idx_v[pl.ds(i, _L)]
            out_v[pl.ds(i, _L)] = plsc.load_gather(x_v, [iv])

    di0.wait()
    dx0.wait()
    gather_row(x0_v, i0_v, o0_v)
    do0 = pltpu.async_copy(o0_v, out_hbm.at[row0], sem_o)
    di1.wait()
    dx1.wait()
    gather_row(x1_v, i1_v, o1_v)
    do1 = pltpu.async_copy(o1_v, out_hbm.at[row1], sem_o)
    do0.wait()
    do1.wait()


@jax.jit
def _run(x, idx):
    mesh = plsc.VectorSubcoreMesh(core_axis_name="c", subcore_axis_name="s")
    f = pl.kernel(
        _gather_body,
        out_type=jax.ShapeDtypeStruct((_B, _K), jnp.float32),
        mesh=mesh,
        scratch_types=[
            pltpu.VMEM((_N,), jnp.float32),
            pltpu.VMEM((_N,), jnp.float32),
            pltpu.VMEM((_K,), jnp.int32),
            pltpu.VMEM((_K,), jnp.int32),
            pltpu.VMEM((_K,), jnp.float32),
            pltpu.VMEM((_K,), jnp.float32),
            pltpu.SemaphoreType.DMA,
            pltpu.SemaphoreType.DMA,
            pltpu.SemaphoreType.DMA,
        ],
        compiler_params=pltpu.CompilerParams(needs_layout_passes=False),
    )
    return f(x, idx)


def kernel(x, assessment, index):
    del assessment
    return _run(x, index.astype(jnp.int32))


# speedup vs baseline: 1.0881x; 1.0019x over previous
import jax
import jax.numpy as jnp
from jax import lax
from jax.experimental import pallas as pl
from jax.experimental.pallas import tpu as pltpu
from jax.experimental.pallas import tpu_sc as plsc

_B, _N, _K = 64, 32768, 2048
_NC, _NS = 2, 16
_NW = _NC * _NS
_RW = _B // _NW
_L = 16


def _gather_body(x_hbm, idx_hbm, out_hbm, x_v, i_v, o_v, sem_a, sem_b):
    wid = lax.axis_index("s") * _NC + lax.axis_index("c")
    row0 = wid * _RW
    row1 = row0 + 1

    di0 = pltpu.async_copy(idx_hbm.at[row0], i_v.at[pl.ds(0, _K)], sem_a)
    dx0 = pltpu.async_copy(x_hbm.at[row0], x_v.at[pl.ds(0, _N)], sem_a)
    di1 = pltpu.async_copy(idx_hbm.at[row1], i_v.at[pl.ds(_K, _K)], sem_b)
    dx1 = pltpu.async_copy(x_hbm.at[row1], x_v.at[pl.ds(_N, _N)], sem_b)

    def gather_row(base, bias):
        @plsc.parallel_loop(0, _K, step=_L, unroll=16)
        def step(i):
            iv = i_v[pl.ds(base + i, _L)]
            o_v[pl.ds(base + i, _L)] = plsc.load_gather(x_v, [iv + bias] if bias else [iv])

    di0.wait()
    dx0.wait()
    gather_row(0, 0)
    do0 = pltpu.async_copy(o_v.at[pl.ds(0, _K)], out_hbm.at[row0], sem_a)
    di1.wait()
    dx1.wait()
    gather_row(_K, _N)
    do1 = pltpu.async_copy(o_v.at[pl.ds(_K, _K)], out_hbm.at[row1], sem_b)
    do0.wait()
    do1.wait()


@jax.jit
def _run(x, idx):
    mesh = plsc.VectorSubcoreMesh(core_axis_name="c", subcore_axis_name="s")
    f = pl.kernel(
        _gather_body,
        out_type=jax.ShapeDtypeStruct((_B, _K), jnp.float32),
        mesh=mesh,
        scratch_types=[
            pltpu.VMEM((_RW * _N,), jnp.float32),
            pltpu.VMEM((_RW * _K,), jnp.int32),
            pltpu.VMEM((_RW * _K,), jnp.float32),
            pltpu.SemaphoreType.DMA,
            pltpu.SemaphoreType.DMA,
        ],
        compiler_params=pltpu.CompilerParams(needs_layout_passes=False),
    )
    return f(x, idx)


def kernel(x, assessment, index):
    del assessment
    return _run(x, index.astype(jnp.int32))
